# baseline (speedup 1.0000x reference)
import jax
import jax.numpy as jnp
import numpy as np
from jax import lax
from jax.experimental import pallas as pl
from jax.experimental.pallas import tpu as pltpu
from jax.experimental.pallas import tpu_sc as plsc

BS = 32
N = 8192
D = 256
AD = 11
K = 900
L = 16
NV = N // L
KV = K // L + 1
CH = 64
NCH = K // CH
REM = K - NCH * CH
CHUNKS = [(c * CH, CH) for c in range(NCH)] + [(NCH * CH, REM)]
MSB = np.int32(-2147483648)


def _desc_key(bits):
    asc = jnp.where(bits < 0, ~bits, bits | MSB)
    return ~asc


def _key_to_f32(key):
    asc = ~key
    bits = jnp.where(asc < 0, asc & np.int32(0x7FFFFFFF), ~asc)
    return plsc.bitcast(bits, jnp.float32)


def _digit(key, shift):
    return lax.shift_right_logical(key, np.int32(shift)) & np.int32(255)


def _sc_body(conf_hbm, feat_hbm, anch_hbm, koff_hbm,
             conf_out, feat_out, anch_out,
             row_v, key0, key1, idx0, idx1, hist, base, vals, gidx, koff_v,
             bidx0, bidx1, oarr, asout,
             fbuf0, fbuf1, abuf0, abuf1, ftail, atail0, atail1,
             fsem0, fsem1, asem0, asem1, fsem2):
    b = lax.axis_index("s") * 2 + lax.axis_index("c")

    pltpu.sync_copy(conf_hbm.at[b], row_v)
    pltpu.sync_copy(koff_hbm, koff_v)

    zeros = jnp.zeros((L,), jnp.int32)
    for j in range(256 // L):
        hist[pl.ds(j * L, L)] = zeros

    def build(r, _):
        for j in range(8):
            bits = plsc.bitcast(row_v[r, pl.ds(j * L, L)], jnp.int32)
            key = _desc_key(bits)
            sl = pl.ds(r * 128 + j * L, L)
            key0[sl] = key
            idx0[sl] = (r * 128 + j * L) + lax.iota(jnp.int32, L)
            d = _digit(key, 0)
            cnt, last = plsc.scan_count(d)
            plsc.addupdate_scatter(hist, [d], cnt, mask=last)
        return 0

    lax.fori_loop(0, N // 128, build, 0)

    def prefix_and_zero():
        carry = jnp.int32(0)
        for j in range(256 // L):
            sl = pl.ds(j * L, L)
            c = hist[sl]
            incl = plsc.cumsum(c)
            base[sl] = incl - c + carry
            hist[sl] = zeros
            carry = carry + jnp.sum(c)

    def radix_pass(src_k, src_i, dst_k, dst_i, shift, next_shift):
        prefix_and_zero()

        def step(i, _):
            sl = pl.ds(i * L, L)
            key = src_k[sl]
            val = src_i[sl]
            d = _digit(key, shift)
            cnt, last = plsc.scan_count(d)
            pos = plsc.load_gather(base, [d]) + cnt - 1
            plsc.store_scatter(dst_k, [pos], key)
            plsc.store_scatter(dst_i, [pos], val)
            plsc.addupdate_scatter(base, [d], cnt, mask=last)
            if next_shift is not None:
                d2 = _digit(key, next_shift)
                cnt2, last2 = plsc.scan_count(d2)
                plsc.addupdate_scatter(hist, [d2], cnt2, mask=last2)
            return 0

        lax.fori_loop(0, NV, step, 0)

    radix_pass(key0, idx0, key1, idx1, 0, 8)
    radix_pass(key1, idx1, key0, idx0, 8, 16)
    radix_pass(key0, idx0, key1, idx1, 16, 24)
    radix_pass(key1, idx1, key0, idx0, 24, None)

    off = koff_v[pl.ds(0, L)] + b * N
    for i in range(KV):
        sl = pl.ds(i * L, L)
        r4, c4 = i // 4, (i % 4) * L
        vals[i // 8, pl.ds((i % 8) * L, L)] = _key_to_f32(key0[sl])
        gi = idx0[sl] + off
        gidx[r4, pl.ds(c4, L)] = gi
        e0 = gi * np.int32(AD)
        bidx0[r4, pl.ds(c4, L)] = lax.shift_right_logical(e0, np.int32(7))
        bidx1[r4, pl.ds(c4, L)] = lax.shift_right_logical(
            e0 + np.int32(AD - 1), np.int32(7))
        oarr[r4, pl.ds(c4, L)] = e0 & np.int32(127)
    pltpu.sync_copy(vals, conf_out.at[b])

    fb = (fbuf0, fbuf1)
    fs = (fsem0, fsem1)
    iota = lax.iota(jnp.int32, L)

    def anchor_extract(b0, b1, ci, sz):
        def one_row(r, ovec, rr):
            o = ovec[rr]
            c0 = o + iota
            m0 = c0 < np.int32(128)
            rsp = jnp.full((L,), r, jnp.int32)
            g0 = plsc.load_gather(b0, [rsp, jnp.minimum(c0, 127)], mask=m0)
            g1 = plsc.load_gather(b1, [rsp, jnp.maximum(c0 - 128, 0)],
                                  mask=~m0)
            asout[r, pl.ds(0, L)] = jnp.where(m0, g0, g1)

        if sz < L:
            ovec = oarr[ci, pl.ds(0, L)]
            for r in range(sz):
                one_row(r, ovec, r)
        else:
            def grp(v, _):
                ovec = oarr[ci, pl.ds(v * L, L)]
                for rr in range(L):
                    one_row(v * L + rr, ovec, rr)
                return 0

            lax.fori_loop(0, sz // L, grp, 0)

    def copy_out(ci, bi):
        off_, sz = CHUNKS[ci]
        rows = pl.ds(off_, sz)
        pltpu.sync_copy(fb[bi].at[pl.ds(0, sz)], feat_out.at[b, rows])
        a0 = pltpu.async_copy(anch_hbm.at[bidx0.at[ci]], abuf0, asem0)
        a1 = pltpu.async_copy(anch_hbm.at[bidx1.at[ci]], abuf1, asem1)
        a0.wait()
        a1.wait()
        anchor_extract(abuf0, abuf1, ci, sz)
        pltpu.sync_copy(asout.at[pl.ds(0, sz)], anch_out.at[b, rows])

    pend = [None, None]
    for ci in range(NCH):
        bi = ci % 2
        if pend[bi] is not None:
            pci, pf = pend[bi]
            pf.wait()
            copy_out(pci, bi)
        fcp = pltpu.async_copy(feat_hbm.at[gidx.at[ci]], fb[bi], fs[bi])
        pend[bi] = (ci, fcp)
    tidx = gidx.at[NCH, pl.ds(0, 8)]
    tf = pltpu.async_copy(feat_hbm.at[tidx], ftail, fsem2)
    for bi in (NCH % 2, (NCH + 1) % 2):
        pci, pf = pend[bi]
        pf.wait()
        copy_out(pci, bi)
    tf.wait()
    rows = pl.ds(NCH * CH, REM)
    pltpu.sync_copy(ftail.at[pl.ds(0, REM)], feat_out.at[b, rows])
    ta0 = pltpu.async_copy(anch_hbm.at[bidx0.at[NCH, pl.ds(0, 8)]],
                           atail0, asem0)
    ta1 = pltpu.async_copy(anch_hbm.at[bidx1.at[NCH, pl.ds(0, 8)]],
                           atail1, asem1)
    ta0.wait()
    ta1.wait()
    anchor_extract(atail0, atail1, NCH, REM)
    pltpu.sync_copy(asout.at[pl.ds(0, REM)], anch_out.at[b, rows])


@jax.jit
def _run(conf3d, feat_flat, anch_dense, koff_arr):
    mesh = plsc.VectorSubcoreMesh(core_axis_name="c", subcore_axis_name="s")
    out_type = (
        jax.ShapeDtypeStruct((BS, 8, 128), jnp.float32),
        jax.ShapeDtypeStruct((BS, K, D), jnp.float32),
        jax.ShapeDtypeStruct((BS, K, 128), jnp.float32),
    )
    scratch = [
        pltpu.VMEM((N // 128, 128), jnp.float32),
        pltpu.VMEM((N,), jnp.int32),
        pltpu.VMEM((N,), jnp.int32),
        pltpu.VMEM((N,), jnp.int32),
        pltpu.VMEM((N,), jnp.int32),
        pltpu.VMEM((256,), jnp.int32),
        pltpu.VMEM((256,), jnp.int32),
        pltpu.VMEM((8, 128), jnp.float32),
        pltpu.VMEM((16, CH), jnp.int32),
        pltpu.VMEM((L,), jnp.int32),
        pltpu.VMEM((16, CH), jnp.int32),
        pltpu.VMEM((16, CH), jnp.int32),
        pltpu.VMEM((16, CH), jnp.int32),
        pltpu.VMEM((CH, 128), jnp.float32),
        pltpu.VMEM((CH, D), jnp.float32),
        pltpu.VMEM((CH, D), jnp.float32),
        pltpu.VMEM((CH, 128), jnp.float32),
        pltpu.VMEM((CH, 128), jnp.float32),
        pltpu.VMEM((8, D), jnp.float32),
        pltpu.VMEM((8, 128), jnp.float32),
        pltpu.VMEM((8, 128), jnp.float32),
        pltpu.SemaphoreType.DMA,
        pltpu.SemaphoreType.DMA,
        pltpu.SemaphoreType.DMA,
        pltpu.SemaphoreType.DMA,
        pltpu.SemaphoreType.DMA,
    ]
    f = pl.kernel(_sc_body, out_type=out_type, mesh=mesh,
                  scratch_types=scratch,
                  compiler_params=pltpu.CompilerParams(
                      needs_layout_passes=False))
    return f(conf3d, feat_flat, anch_dense, koff_arr)


def kernel(confidence, instance_feature, anchor, k):
    koff = jnp.asarray(k, jnp.int32) - np.int32(K)
    koff_arr = jnp.full((L,), koff, jnp.int32)
    conf3d = confidence.reshape(BS, N // 128, 128)
    feat_flat = instance_feature.reshape(BS * N, D)
    anch_dense = anchor.reshape(BS * N * AD // 128, 128)
    conf, feat, anch = _run(conf3d, feat_flat, anch_dense, koff_arr)
    return (conf.reshape(BS, 1024)[:, :K], feat, anch[:, :, :AD])


# speedup vs baseline: 1.2429x; 1.2429x over previous
import jax
import jax.numpy as jnp
import numpy as np
from jax import lax
from jax.experimental import pallas as pl
from jax.experimental.pallas import tpu as pltpu
from jax.experimental.pallas import tpu_sc as plsc

BS = 32
N = 8192
D = 256
AD = 11
K = 900
L = 16
KV = K // L + 1
CH = 64
NCH = K // CH
REM = K - NCH * CH
CHUNKS = [(c * CH, CH) for c in range(NCH)] + [(NCH * CH, REM)]
CB = N + L
NBINS = 2048
MSB = np.int32(-2147483648)


def _desc_key(bits):
    asc = jnp.where(bits < 0, ~bits, bits | MSB)
    return ~asc


def _key_to_f32(key):
    asc = ~key
    bits = jnp.where(asc < 0, asc & np.int32(0x7FFFFFFF), ~asc)
    return plsc.bitcast(bits, jnp.float32)


def _digit(key, shift, mask):
    return lax.shift_right_logical(key, np.int32(shift)) & np.int32(mask)


def _sc_body(conf_hbm, feat_hbm, anch_hbm, koff_hbm,
             conf_out, feat_out, anch_out,
             row_v, ckey0, ckey1, cidx0, cidx1, hist, base, vals, gidx,
             koff_v, fbuf0, fbuf1, abuf0, abuf1, ftail, atail,
             fsem0, fsem1, asem0, asem1, fsem2, asem2):
    b = lax.axis_index("s") * 2 + lax.axis_index("c")
    iota = lax.iota(jnp.int32, L)
    zeros = jnp.zeros((L,), jnp.int32)

    pltpu.sync_copy(conf_hbm.at[b], row_v)
    pltpu.sync_copy(koff_hbm, koff_v)

    def zero_hist(g, _):
        hist[pl.ds(g * L, L)] = zeros
        return 0

    lax.fori_loop(0, NBINS // L, zero_hist, 0)

    def build(r, _):
        for j in range(8):
            bits = plsc.bitcast(row_v[r, pl.ds(j * L, L)], jnp.int32)
            d = _digit(_desc_key(bits), 21, 2047)
            cnt, last = plsc.scan_count(d)
            plsc.addupdate_scatter(hist, [d], cnt, mask=last)
        return 0

    lax.fori_loop(0, N // 128, build, 0)

    def cutfind(g, carry):
        tot, cut = carry
        sl = pl.ds(g * L, L)
        c = hist[sl]
        incl = plsc.cumsum(c) + tot
        cand = jnp.where(incl >= np.int32(K), g * L + iota, np.int32(NBINS))
        hist[sl] = zeros
        return tot + jnp.sum(c), jnp.minimum(cut, jnp.min(cand))

    _, cut = lax.fori_loop(0, NBINS // L, cutfind,
                           (jnp.int32(0), jnp.int32(NBINS)))

    def compact(r, off):
        for j in range(8):
            bits = plsc.bitcast(row_v[r, pl.ds(j * L, L)], jnp.int32)
            key = _desc_key(bits)
            m = _digit(key, 21, 2047) <= cut
            mi = m.astype(jnp.int32)
            pos = off + plsc.cumsum(mi) - 1
            plsc.store_scatter(ckey0, [pos], key, mask=m)
            plsc.store_scatter(cidx0, [pos], r * 128 + j * L + iota, mask=m)
            d0 = key & np.int32(2047)
            cnt, last = plsc.scan_count(d0, m)
            plsc.addupdate_scatter(hist, [d0], cnt, mask=last & m)
            off = off + jnp.sum(mi)
        return off

    nsur = lax.fori_loop(0, N // 128, compact, jnp.int32(0))
    plsc.store_scatter(ckey0, [nsur + iota], jnp.full((L,), -1, jnp.int32))
    plsc.store_scatter(cidx0, [nsur + iota], zeros)
    nv = lax.shift_right_logical(nsur + np.int32(L - 1), np.int32(4))
    npad = nv * L - nsur
    plsc.addupdate_scatter(hist, [jnp.full((L,), NBINS - 1, jnp.int32)],
                           jnp.full((L,), npad, jnp.int32),
                           mask=iota == 0)

    def prefix_and_zero(nbins):
        def body(g, carry):
            sl = pl.ds(g * L, L)
            c = hist[sl]
            incl = plsc.cumsum(c)
            base[sl] = incl - c + carry
            hist[sl] = zeros
            return carry + jnp.sum(c)

        lax.fori_loop(0, nbins // L, body, jnp.int32(0))

    def radix_pass(src_k, src_i, dst_k, dst_i, shift, mask, nxt):
        prefix_and_zero(NBINS)

        def step(i, _):
            sl = pl.ds(i * L, L)
            key = src_k[sl]
            val = src_i[sl]
            d = _digit(key, shift, mask)
            cnt, last = plsc.scan_count(d)
            pos = plsc.load_gather(base, [d]) + cnt - 1
            plsc.store_scatter(dst_k, [pos], key)
            plsc.store_scatter(dst_i, [pos], val)
            plsc.addupdate_scatter(base, [d], cnt, mask=last)
            if nxt is not None:
                d2 = _digit(key, nxt[0], nxt[1])
                cnt2, last2 = plsc.scan_count(d2)
                plsc.addupdate_scatter(hist, [d2], cnt2, mask=last2)
            return 0

        lax.fori_loop(0, nv, step, 0)

    radix_pass(ckey0, cidx0, ckey1, cidx1, 0, 2047, (11, 2047))
    radix_pass(ckey1, cidx1, ckey0, cidx0, 11, 2047, (22, 1023))
    radix_pass(ckey0, cidx0, ckey1, cidx1, 22, 1023, None)

    off = koff_v[pl.ds(0, L)] + b * N
    for i in range(KV):
        sl = pl.ds(i * L, L)
        vals[i // 8, pl.ds((i % 8) * L, L)] = _key_to_f32(ckey1[sl])
        gidx[i // 4, pl.ds((i % 4) * L, L)] = cidx1[sl] + off
    pltpu.sync_copy(vals, conf_out.at[b])

    fb, ab = (fbuf0, fbuf1), (abuf0, abuf1)
    fs, asms = (fsem0, fsem1), (asem0, asem1)

    def copy_out(ci, bi):
        off_, sz = CHUNKS[ci]
        rows = pl.ds(off_, sz)
        pltpu.sync_copy(fb[bi].at[pl.ds(0, sz)], feat_out.at[b, rows])
        pltpu.sync_copy(ab[bi].at[pl.ds(0, sz)], anch_out.at[b, rows])

    pend = [None, None]
    for ci in range(NCH):
        bi = ci % 2
        if pend[bi] is not None:
            pci, pf, pa = pend[bi]
            pf.wait()
            pa.wait()
            copy_out(pci, bi)
        fcp = pltpu.async_copy(feat_hbm.at[gidx.at[ci]], fb[bi], fs[bi])
        acp = pltpu.async_copy(anch_hbm.at[gidx.at[ci]], ab[bi], asms[bi])
        pend[bi] = (ci, fcp, acp)
    tidx = gidx.at[NCH, pl.ds(0, 8)]
    tf = pltpu.async_copy(feat_hbm.at[tidx], ftail, fsem2)
    ta = pltpu.async_copy(anch_hbm.at[tidx], atail, asem2)
    for bi in (NCH % 2, (NCH + 1) % 2):
        pci, pf, pa = pend[bi]
        pf.wait()
        pa.wait()
        copy_out(pci, bi)
    tf.wait()
    ta.wait()
    rows = pl.ds(NCH * CH, REM)
    pltpu.sync_copy(ftail.at[pl.ds(0, REM)], feat_out.at[b, rows])
    pltpu.sync_copy(atail.at[pl.ds(0, REM)], anch_out.at[b, rows])


@jax.jit
def _run(conf3d, feat_flat, anch128, koff_arr):
    mesh = plsc.VectorSubcoreMesh(core_axis_name="c", subcore_axis_name="s")
    out_type = (
        jax.ShapeDtypeStruct((BS, 8, 128), jnp.float32),
        jax.ShapeDtypeStruct((BS, K, D), jnp.float32),
        jax.ShapeDtypeStruct((BS, K, 128), jnp.float32),
    )
    scratch = [
        pltpu.VMEM((N // 128, 128), jnp.float32),
        pltpu.VMEM((CB,), jnp.int32),
        pltpu.VMEM((CB,), jnp.int32),
        pltpu.VMEM((CB,), jnp.int32),
        pltpu.VMEM((CB,), jnp.int32),
        pltpu.VMEM((NBINS,), jnp.int32),
        pltpu.VMEM((NBINS,), jnp.int32),
        pltpu.VMEM((8, 128), jnp.float32),
        pltpu.VMEM((16, CH), jnp.int32),
        pltpu.VMEM((L,), jnp.int32),
        pltpu.VMEM((CH, D), jnp.float32),
        pltpu.VMEM((CH, D), jnp.float32),
        pltpu.VMEM((CH, 128), jnp.float32),
        pltpu.VMEM((CH, 128), jnp.float32),
        pltpu.VMEM((8, D), jnp.float32),
        pltpu.VMEM((8, 128), jnp.float32),
        pltpu.SemaphoreType.DMA,
        pltpu.SemaphoreType.DMA,
        pltpu.SemaphoreType.DMA,
        pltpu.SemaphoreType.DMA,
        pltpu.SemaphoreType.DMA,
        pltpu.SemaphoreType.DMA,
    ]
    f = pl.kernel(_sc_body, out_type=out_type, mesh=mesh,
                  scratch_types=scratch,
                  compiler_params=pltpu.CompilerParams(
                      needs_layout_passes=False))
    return f(conf3d, feat_flat, anch128, koff_arr)


def kernel(confidence, instance_feature, anchor, k):
    koff = jnp.asarray(k, jnp.int32) - np.int32(K)
    koff_arr = jnp.full((L,), koff, jnp.int32)
    conf3d = confidence.reshape(BS, N // 128, 128)
    feat_flat = instance_feature.reshape(BS * N, D)
    anch128 = jnp.pad(anchor.reshape(BS * N, AD), ((0, 0), (0, 128 - AD)))
    conf, feat, anch = _run(conf3d, feat_flat, anch128, koff_arr)
    return (conf.reshape(BS, 1024)[:, :K], feat, anch[:, :, :AD])
